# Initial kernel scaffold; baseline (speedup 1.0000x reference)
import jax
import jax.numpy as jnp
from jax.experimental import pallas as pl


def kernel(input, token_table, pos_table):
    raise NotImplementedError("write your pallas kernel here")


# speedup vs baseline: 1.4216x; 1.4216x over previous
import functools

import jax
import jax.numpy as jnp
from jax import lax
from jax.experimental import pallas as pl
from jax.experimental.pallas import tpu as pltpu
from jax.experimental.pallas import tpu_sc as plsc

NC = 2
NS = 16
NW = NC * NS

LANES = 16
L_SEQ = 200
SEQ_PER_CHUNK = 8
CHUNK = SEQ_PER_CHUNK * L_SEQ
GATHER = 100
G_PER_CHUNK = CHUNK // GATHER


@functools.lru_cache(maxsize=None)
def _emb_call(n_rows: int, d: int):
    assert d == 2 * LANES
    n_per_w = n_rows // NW
    assert n_per_w * NW == n_rows and n_per_w % CHUNK == 0
    n_chunks = n_per_w // CHUNK

    mesh = plsc.VectorSubcoreMesh(
        core_axis_name="c", subcore_axis_name="s",
        num_cores=NC, num_subcores=NS)

    @functools.partial(
        pl.kernel,
        out_type=jax.ShapeDtypeStruct((n_rows, d), jnp.float32),
        mesh=mesh,
        scratch_types=[
            pltpu.VMEM((G_PER_CHUNK, GATHER), jnp.int32),
            pltpu.VMEM((CHUNK, d), jnp.float32),
            pltpu.VMEM((L_SEQ, d), jnp.float32),
            pltpu.SemaphoreType.DMA,
        ],
        compiler_params=pltpu.CompilerParams(use_tc_tiling_on_sc=False),
    )
    def run(idx_hbm, table_hbm, pos_hbm, out_hbm, idx_v, rows_v, pos_v, sem):
        wid = lax.axis_index("s") * NC + lax.axis_index("c")
        pltpu.sync_copy(pos_hbm, pos_v)

        def chunk_body(g, carry):
            base = pl.multiple_of(wid * n_per_w + g * CHUNK, CHUNK)
            irow = pl.multiple_of(
                (wid * n_per_w + g * CHUNK) // GATHER, G_PER_CHUNK)
            pltpu.sync_copy(idx_hbm.at[pl.ds(irow, G_PER_CHUNK)], idx_v)
            cps = [
                pltpu.async_copy(table_hbm.at[idx_v.at[j]],
                                 rows_v.at[pl.ds(j * GATHER, GATHER)], sem)
                for j in range(G_PER_CHUNK)
            ]
            for cp in cps:
                cp.wait()

            def add_body(l, c):
                p0 = pos_v[l, pl.ds(0, LANES)]
                p1 = pos_v[l, pl.ds(LANES, LANES)]
                for s in range(SEQ_PER_CHUNK):
                    r = s * L_SEQ + l
                    rows_v[r, pl.ds(0, LANES)] = rows_v[r, pl.ds(0, LANES)] + p0
                    rows_v[r, pl.ds(LANES, LANES)] = (
                        rows_v[r, pl.ds(LANES, LANES)] + p1)
                return c

            lax.fori_loop(0, L_SEQ, add_body, 0)
            pltpu.sync_copy(rows_v, out_hbm.at[pl.ds(base, CHUNK)])
            return carry

        lax.fori_loop(0, n_chunks, chunk_body, 0)

    return run


def kernel(input, token_table, pos_table):
    b, l = input.shape
    v, d = token_table.shape
    idx = input.reshape(b * l // GATHER, GATHER).astype(jnp.int32)
    out = _emb_call(b * l, d)(idx, token_table, pos_table.astype(jnp.float32))
    return out.reshape(b, l, d)
